# baseline (speedup 1.0000x reference)
import functools

import jax
import jax.numpy as jnp
from jax import lax
from jax.experimental import pallas as pl
from jax.experimental.pallas import tpu as pltpu
from jax.experimental.pallas import tpu_sc as plsc

NUM_EMB = 1000000
D = 32
T = 50
NC = 2
NS = 16
NW = NC * NS
NBLK = 16384 // 128
BPW = NBLK // NW
NUNIT = T * BPW
L = 16


def _make_kernel():
  mesh = plsc.VectorSubcoreMesh(core_axis_name="c", subcore_axis_name="s")

  @functools.partial(
      pl.kernel,
      out_type=jax.ShapeDtypeStruct((T, 4, 128, 8, 128), jnp.float32),
      mesh=mesh,
      compiler_params=pltpu.CompilerParams(
          use_tc_tiling_on_sc=False, needs_layout_passes=False
      ),
      scratch_types=[
          pltpu.VMEM((T, BPW, 128), jnp.int32),
          [pltpu.VMEM((128, D), jnp.float32) for _ in range(2)],
          [pltpu.VMEM((4, 8, 128), jnp.float32) for _ in range(2)],
          [pltpu.SemaphoreType.DMA for _ in range(2)],
          [pltpu.SemaphoreType.DMA for _ in range(2)],
      ],
  )
  def emb_kernel(idx_hbm, table_hbm, out_hbm, idx_v, rbufs, tbufs, gsems,
                 osems):
    wid = lax.axis_index("s") * NC + lax.axis_index("c")
    pltpu.sync_copy(idx_hbm.at[:, pl.ds(wid * BPW, BPW), :], idx_v)

    iota = lax.iota(jnp.int32, L)

    def fire_gather(u, b):
      t = u // BPW
      jb = u % BPW
      pltpu.async_copy(table_hbm.at[idx_v.at[t, jb]], rbufs[b], gsems[b])

    def drain_gather(b):
      pltpu.make_async_copy(
          table_hbm.at[pl.ds(0, 128)], rbufs[b], gsems[b]
      ).wait()

    def transpose_unit(b):
      rbuf = rbufs[b]
      tbuf = tbufs[b]
      for ct in range(4):
        for s in range(8):
          col = jnp.full((L,), 8 * ct + s, jnp.int32)
          for l0 in range(0, 128, L):
            v = plsc.load_gather(rbuf, [iota + l0, col])
            tbuf[ct, s, pl.ds(l0, L)] = v

    def fire_out(u, b):
      t = u // BPW
      jb = u % BPW
      pltpu.async_copy(
          tbufs[b], out_hbm.at[t, :, wid * BPW + jb], osems[b]
      )

    def drain_out(b):
      pltpu.make_async_copy(
          tbufs[b], out_hbm.at[0, :, 0], osems[b]
      ).wait()

    fire_gather(0, 0)

    def body(g, carry):
      for b in range(2):
        u = g * 2 + b
        nb = 1 - b
        drain_gather(b)

        @pl.when(u + 1 < NUNIT)
        def _():
          fire_gather(u + 1, nb)

        @pl.when(u >= 2)
        def _():
          drain_out(b)

        transpose_unit(b)
        fire_out(u, b)
      return carry

    lax.fori_loop(0, NUNIT // 2, body, 0)
    drain_out(0)
    drain_out(1)

  return emb_kernel


_emb = _make_kernel()


@jax.jit
def kernel(token_idx, weight):
  idx = token_idx.T.reshape(T, NBLK, 128)
  out5 = _emb(idx, weight)
  return out5.transpose(2, 4, 0, 1, 3).reshape(16384, T, D)


# speedup vs baseline: 1.1908x; 1.1908x over previous
import functools

import jax
import jax.numpy as jnp
from jax import lax
from jax.experimental import pallas as pl
from jax.experimental.pallas import tpu as pltpu
from jax.experimental.pallas import tpu_sc as plsc

NUM_EMB = 1000000
D = 32
T = 50
NC = 2
NS = 16
NW = NC * NS
NBLK = 16384 // 128
BPW = NBLK // NW
NUNIT = T * BPW
L = 16


def _make_kernel():
  mesh = plsc.VectorSubcoreMesh(core_axis_name="c", subcore_axis_name="s")

  @functools.partial(
      pl.kernel,
      out_type=jax.ShapeDtypeStruct((T, 4, 128, 8, 128), jnp.float32),
      mesh=mesh,
      compiler_params=pltpu.CompilerParams(
          use_tc_tiling_on_sc=False, needs_layout_passes=False
      ),
      scratch_types=[
          pltpu.VMEM((T, BPW, 128), jnp.int32),
          [pltpu.VMEM((128, D), jnp.float32) for _ in range(2)],
          [pltpu.VMEM((4, 8, 128), jnp.float32) for _ in range(2)],
          [pltpu.SemaphoreType.DMA for _ in range(2)],
          [pltpu.SemaphoreType.DMA for _ in range(2)],
      ],
  )
  def emb_kernel(idx_hbm, table_hbm, out_hbm, idx_v, rbufs, tbufs, gsems,
                 osems):
    wid = lax.axis_index("s") * NC + lax.axis_index("c")
    pltpu.sync_copy(idx_hbm.at[:, pl.ds(wid * BPW, BPW), :], idx_v)

    iota = lax.iota(jnp.int32, L)

    def fire_gather(u, b):
      t = u // BPW
      jb = u % BPW
      pltpu.async_copy(table_hbm.at[idx_v.at[t, jb]], rbufs[b], gsems[b])

    def drain_gather(b):
      pltpu.make_async_copy(
          table_hbm.at[pl.ds(0, 128)], rbufs[b], gsems[b]
      ).wait()

    def transpose_unit(b):
      rbuf = rbufs[b]
      tbuf = tbufs[b]
      for ct in range(4):
        for s in range(8):
          col = jnp.full((L,), 8 * ct + s, jnp.int32)
          vs = [
              plsc.load_gather(rbuf, [iota + l0, col])
              for l0 in range(0, 128, L)
          ]
          for k in range(8):
            tbuf[ct, s, pl.ds(k * L, L)] = vs[k]

    def fire_out(u, b):
      t = u // BPW
      jb = u % BPW
      pltpu.async_copy(
          tbufs[b], out_hbm.at[t, :, wid * BPW + jb], osems[b]
      )

    def drain_out(b):
      pltpu.make_async_copy(
          tbufs[b], out_hbm.at[0, :, 0], osems[b]
      ).wait()

    fire_gather(0, 0)

    def body(g, carry):
      for b in range(2):
        u = g * 2 + b
        nb = 1 - b
        drain_gather(b)

        @pl.when(u + 1 < NUNIT)
        def _():
          fire_gather(u + 1, nb)

        @pl.when(u >= 2)
        def _():
          drain_out(b)

        transpose_unit(b)
        fire_out(u, b)
      return carry

    lax.fori_loop(0, NUNIT // 2, body, 0)
    drain_out(0)
    drain_out(1)

  return emb_kernel


_emb = _make_kernel()


@jax.jit
def kernel(token_idx, weight):
  idx = token_idx.T.reshape(T, NBLK, 128)
  out5 = _emb(idx, weight)
  return out5.transpose(2, 4, 0, 1, 3).reshape(16384, T, D)
